# baseline (speedup 1.0000x reference)
import functools

import jax
import jax.numpy as jnp
from jax import lax
from jax.experimental import pallas as pl
from jax.experimental.pallas import tpu as pltpu
from jax.experimental.pallas import tpu_sc as plsc

N = 32768
D = 2048
L = 16
NC = 2
NS = 16
NW = NC * NS
ROWS_PER_W = N // NW
CH = 8
OCH = 16
NCHUNK = ROWS_PER_W // CH
NOCHUNK = ROWS_PER_W // OCH
JBLK = D // L
NBUF = 2
UF = 4


def _body(x_hbm, perm_hbm, out_hbm, perm_v, in_bufs, out_bufs, in_sems,
          out_sems):
    wid = lax.axis_index("s") * NC + lax.axis_index("c")
    base = wid * ROWS_PER_W

    pltpu.sync_copy(perm_hbm, perm_v)

    def in_slice(g):
        return x_hbm.at[pl.ds(base + g * CH, CH)]

    def out_slice(G):
        return out_hbm.at[pl.ds(base + G * OCH, OCH)]

    for b in range(NBUF):
        pltpu.async_copy(in_slice(b), in_bufs[b], in_sems[b])

    def quad(p, _):
        for b in range(UF):
            g = p * UF + b
            G = g // 2
            ib = b % NBUF
            ob = b // 2
            h = b % 2
            pltpu.make_async_copy(in_slice(g), in_bufs[ib], in_sems[ib]).wait()
            if h == 0:
                @pl.when(G >= NBUF)
                def _():
                    pltpu.make_async_copy(
                        out_bufs[ob], out_slice(G - NBUF), out_sems[ob]).wait()

            @plsc.parallel_loop(0, JBLK, unroll=4)
            def _(j):
                col = perm_v[pl.ds(j * L, L)]
                for r in range(CH):
                    rows = jnp.full((L,), r, jnp.int32)
                    out_bufs[ob][h * CH + r, pl.ds(j * L, L)] = (
                        plsc.load_gather(in_bufs[ib], [rows, col]))

            if h == 1:
                pltpu.async_copy(out_bufs[ob], out_slice(G), out_sems[ob])

            @pl.when(g + NBUF < NCHUNK)
            def _():
                pltpu.async_copy(
                    in_slice(g + NBUF), in_bufs[ib], in_sems[ib])
        return 0

    lax.fori_loop(0, NCHUNK // UF, quad, 0)

    for b in range(NBUF):
        G = NOCHUNK - NBUF + b
        ob = G % NBUF
        pltpu.make_async_copy(out_bufs[ob], out_slice(G), out_sems[ob]).wait()


@jax.jit
def kernel(x, perm):
    mesh = plsc.VectorSubcoreMesh(core_axis_name="c", subcore_axis_name="s")
    run = pl.kernel(
        _body,
        out_type=jax.ShapeDtypeStruct((N, D), jnp.float32),
        mesh=mesh,
        compiler_params=pltpu.CompilerParams(needs_layout_passes=False),
        scratch_types=[
            pltpu.VMEM((D,), jnp.int32),
            [pltpu.VMEM((CH, D), jnp.float32) for _ in range(NBUF)],
            [pltpu.VMEM((OCH, D), jnp.float32) for _ in range(NBUF)],
            [pltpu.SemaphoreType.DMA for _ in range(NBUF)],
            [pltpu.SemaphoreType.DMA for _ in range(NBUF)],
        ],
    )
    return run(x, perm)


# speedup vs baseline: 1.0275x; 1.0275x over previous
import jax
import jax.numpy as jnp
from jax import lax
from jax.experimental import pallas as pl
from jax.experimental.pallas import tpu as pltpu
from jax.experimental.pallas import tpu_sc as plsc

N = 32768
D = 2048
L = 16
NC = 2
NS = 16
NW = NC * NS
ROWS_PER_W = N // NW
CH = 16
HR = 8
NCHUNK = ROWS_PER_W // CH
JBLK = D // L
NBUF = 2


def _body(x_hbm, perm_hbm, out_hbm, perm_v, in_bufs, out_bufs, in_sems,
          out_sems):
    wid = lax.axis_index("s") * NC + lax.axis_index("c")
    base = wid * ROWS_PER_W

    pltpu.sync_copy(perm_hbm, perm_v)

    def in_slice(g):
        return x_hbm.at[pl.ds(base + g * CH, CH)]

    def out_slice(g, h):
        return out_hbm.at[pl.ds(base + g * CH + h * HR, HR)]

    for b in range(NBUF):
        pltpu.async_copy(in_slice(b), in_bufs[b], in_sems[b])

    def pair(p, _):
        for b in range(NBUF):
            g = p * NBUF + b
            pltpu.make_async_copy(in_slice(g), in_bufs[b], in_sems[b]).wait()
            for h in range(2):
                @pl.when(g >= 1)
                def _():
                    pltpu.make_async_copy(
                        out_bufs[h], out_slice(g - 1, h), out_sems[h]).wait()

                @plsc.parallel_loop(0, JBLK, unroll=4)
                def _(j):
                    col = perm_v[pl.ds(j * L, L)]
                    for r in range(HR):
                        rows = jnp.full((L,), h * HR + r, jnp.int32)
                        out_bufs[h][r, pl.ds(j * L, L)] = plsc.load_gather(
                            in_bufs[b], [rows, col])

                pltpu.async_copy(out_bufs[h], out_slice(g, h), out_sems[h])

            @pl.when(g + NBUF < NCHUNK)
            def _():
                pltpu.async_copy(
                    in_slice(g + NBUF), in_bufs[b], in_sems[b])
        return 0

    lax.fori_loop(0, NCHUNK // NBUF, pair, 0)

    for h in range(2):
        pltpu.make_async_copy(
            out_bufs[h], out_slice(NCHUNK - 1, h), out_sems[h]).wait()


@jax.jit
def kernel(x, perm):
    mesh = plsc.VectorSubcoreMesh(core_axis_name="c", subcore_axis_name="s")
    run = pl.kernel(
        _body,
        out_type=jax.ShapeDtypeStruct((N, D), jnp.float32),
        mesh=mesh,
        compiler_params=pltpu.CompilerParams(needs_layout_passes=False),
        scratch_types=[
            pltpu.VMEM((D,), jnp.int32),
            [pltpu.VMEM((CH, D), jnp.float32) for _ in range(NBUF)],
            [pltpu.VMEM((HR, D), jnp.float32) for _ in range(2)],
            [pltpu.SemaphoreType.DMA for _ in range(NBUF)],
            [pltpu.SemaphoreType.DMA for _ in range(2)],
        ],
    )
    return run(x, perm)
